# Initial kernel scaffold; baseline (speedup 1.0000x reference)
import jax
import jax.numpy as jnp
from jax.experimental import pallas as pl


def kernel(x, edge_index, edge_attr, W1, b1, W2, b2):
    raise NotImplementedError("write your pallas kernel here")


# speedup vs baseline: 2.0012x; 2.0012x over previous
import functools

import jax
import jax.numpy as jnp
from jax import lax
from jax.experimental import pallas as pl
from jax.experimental.pallas import tpu as pltpu
from jax.experimental.pallas import tpu_sc as plsc

D_FEAT = 256
HIDDEN = 256
OUT_DIM = 256


_PROJ_BLK = 2000


def _proj_body(x_ref, ws_ref, wd_ref, ys_ref, yd_ref):
    xb = x_ref[...]
    ys_ref[...] = jnp.dot(xb, ws_ref[...], preferred_element_type=jnp.float32)
    yd_ref[...] = jnp.dot(xb, wd_ref[...], preferred_element_type=jnp.float32)


def _project_nodes(x, ws, wd):
    n = x.shape[0]
    grid = n // _PROJ_BLK
    return pl.pallas_call(
        _proj_body,
        grid=(grid,),
        in_specs=[
            pl.BlockSpec((_PROJ_BLK, D_FEAT), lambda i: (i, 0)),
            pl.BlockSpec((D_FEAT, HIDDEN), lambda i: (0, 0)),
            pl.BlockSpec((D_FEAT, HIDDEN), lambda i: (0, 0)),
        ],
        out_specs=[
            pl.BlockSpec((_PROJ_BLK, HIDDEN), lambda i: (i, 0)),
            pl.BlockSpec((_PROJ_BLK, HIDDEN), lambda i: (i, 0)),
        ],
        out_shape=[
            jax.ShapeDtypeStruct((n, HIDDEN), jnp.float32),
            jax.ShapeDtypeStruct((n, HIDDEN), jnp.float32),
        ],
    )(x, ws, wd)


_CH = 128


def _make_gather_add(n_edges):
    info = plsc.get_sparse_core_info()
    nw = info.num_cores * info.num_subcores
    e_per_w = n_edges // nw
    assert e_per_w * nw == n_edges and e_per_w % 8 == 0
    n_full = e_per_w // _CH
    tail = e_per_w - n_full * _CH

    mesh = plsc.VectorSubcoreMesh(core_axis_name="c", subcore_axis_name="s")

    @functools.partial(
        pl.kernel,
        mesh=mesh,
        out_type=jax.ShapeDtypeStruct((n_edges, HIDDEN), jnp.float32),
        scratch_types=[
            pltpu.VMEM((_CH,), jnp.int32),
            pltpu.VMEM((_CH,), jnp.int32),
            pltpu.VMEM((_CH, HIDDEN), jnp.float32),
            pltpu.VMEM((_CH, HIDDEN), jnp.float32),
            pltpu.SemaphoreType.DMA,
            pltpu.SemaphoreType.DMA,
        ],
    )
    def gather_add(ys, yd, src, dst, out, idx_s, idx_d, buf_a, buf_b, sem_a, sem_b):
        wid = lax.axis_index("s") * info.num_cores + lax.axis_index("c")
        base0 = wid * e_per_w

        def do_chunk(base, nrows):
            pltpu.sync_copy(src.at[pl.ds(base, nrows)], idx_s.at[pl.ds(0, nrows)])
            pltpu.sync_copy(dst.at[pl.ds(base, nrows)], idx_d.at[pl.ds(0, nrows)])
            cp_a = pltpu.async_copy(ys.at[idx_s], buf_a, sem_a)
            cp_b = pltpu.async_copy(yd.at[idx_d], buf_b, sem_b)
            cp_a.wait()
            cp_b.wait()

            def add_row(i, carry):
                for j in range(HIDDEN // 16):
                    sl = pl.ds(j * 16, 16)
                    buf_a[i, sl] = buf_a[i, sl] + buf_b[i, sl]
                return carry

            lax.fori_loop(0, nrows, add_row, 0)
            pltpu.sync_copy(buf_a.at[pl.ds(0, nrows)], out.at[pl.ds(base, nrows)])

        def chunk_body(c, carry):
            do_chunk(base0 + c * _CH, _CH)
            return carry

        lax.fori_loop(0, n_full, chunk_body, 0)
        if tail:
            do_chunk(base0 + n_full * _CH, tail)

    return gather_add


_MLP_BLK = 2000


def _mlp_body(g_ref, attr_ref, we_ref, b1_ref, w2_ref, b2_ref, out_ref):
    h = (
        g_ref[...]
        + jnp.dot(attr_ref[...], we_ref[...], preferred_element_type=jnp.float32)
        + b1_ref[...]
    )
    h = jnp.maximum(h, 0.0)
    out_ref[...] = (
        jnp.dot(h, w2_ref[...], preferred_element_type=jnp.float32) + b2_ref[...]
    )


def _edge_mlp(g, edge_attr, we, b1, w2, b2):
    e = g.shape[0]
    d_edge = edge_attr.shape[1]
    grid = e // _MLP_BLK
    return pl.pallas_call(
        _mlp_body,
        grid=(grid,),
        in_specs=[
            pl.BlockSpec((_MLP_BLK, HIDDEN), lambda i: (i, 0)),
            pl.BlockSpec((_MLP_BLK, d_edge), lambda i: (i, 0)),
            pl.BlockSpec((d_edge, HIDDEN), lambda i: (0, 0)),
            pl.BlockSpec((1, HIDDEN), lambda i: (0, 0)),
            pl.BlockSpec((HIDDEN, OUT_DIM), lambda i: (0, 0)),
            pl.BlockSpec((1, OUT_DIM), lambda i: (0, 0)),
        ],
        out_specs=pl.BlockSpec((_MLP_BLK, OUT_DIM), lambda i: (i, 0)),
        out_shape=jax.ShapeDtypeStruct((e, OUT_DIM), jnp.float32),
    )(g, edge_attr, we, b1, w2, b2)


def kernel(x, edge_index, edge_attr, W1, b1, W2, b2):
    d = x.shape[1]
    ws = W1[:d]
    wd = W1[d : 2 * d]
    we = W1[2 * d :]
    src = edge_index[0].astype(jnp.int32)
    dst = edge_index[1].astype(jnp.int32)

    ys, yd = _project_nodes(x, ws, wd)
    g = _make_gather_add(edge_attr.shape[0])(ys, yd, src, dst)
    h_e = _edge_mlp(
        g,
        edge_attr,
        we,
        b1.reshape(1, -1),
        W2,
        b2.reshape(1, -1),
    )
    return (x, edge_index, h_e)


# speedup vs baseline: 2.6460x; 1.3222x over previous
import functools

import jax
import jax.numpy as jnp
from jax import lax
from jax.experimental import pallas as pl
from jax.experimental.pallas import tpu as pltpu
from jax.experimental.pallas import tpu_sc as plsc

D_FEAT = 256
HIDDEN = 256
OUT_DIM = 256


_PROJ_BLK = 2000


def _proj_body(x_ref, ws_ref, wd_ref, ys_ref, yd_ref):
    xb = x_ref[...]
    ys_ref[...] = jnp.dot(xb, ws_ref[...], preferred_element_type=jnp.float32)
    yd_ref[...] = jnp.dot(xb, wd_ref[...], preferred_element_type=jnp.float32)


def _project_nodes(x, ws, wd):
    n = x.shape[0]
    grid = n // _PROJ_BLK
    return pl.pallas_call(
        _proj_body,
        grid=(grid,),
        in_specs=[
            pl.BlockSpec((_PROJ_BLK, D_FEAT), lambda i: (i, 0)),
            pl.BlockSpec((D_FEAT, HIDDEN), lambda i: (0, 0)),
            pl.BlockSpec((D_FEAT, HIDDEN), lambda i: (0, 0)),
        ],
        out_specs=[
            pl.BlockSpec((_PROJ_BLK, HIDDEN), lambda i: (i, 0)),
            pl.BlockSpec((_PROJ_BLK, HIDDEN), lambda i: (i, 0)),
        ],
        out_shape=[
            jax.ShapeDtypeStruct((n, HIDDEN), jnp.float32),
            jax.ShapeDtypeStruct((n, HIDDEN), jnp.float32),
        ],
    )(x, ws, wd)


_CH = 64


def _make_gather_add(n_edges):
    info = plsc.get_sparse_core_info()
    nw = info.num_cores * info.num_subcores
    e_per_w = n_edges // nw
    assert e_per_w * nw == n_edges and e_per_w % 8 == 0
    n_full = e_per_w // _CH
    assert n_full % 2 == 0 and e_per_w % 8 == 0
    has_tail = n_full * _CH < e_per_w

    mesh = plsc.VectorSubcoreMesh(core_axis_name="c", subcore_axis_name="s")

    @functools.partial(
        pl.kernel,
        mesh=mesh,
        out_type=jax.ShapeDtypeStruct((n_edges, HIDDEN), jnp.float32),
        scratch_types=[
            pltpu.VMEM((e_per_w,), jnp.int32),
            pltpu.VMEM((e_per_w,), jnp.int32),
            pltpu.VMEM((2, _CH, HIDDEN), jnp.float32),
            pltpu.VMEM((2, _CH, HIDDEN), jnp.float32),
            pltpu.SemaphoreType.DMA,
            pltpu.SemaphoreType.DMA,
            pltpu.SemaphoreType.DMA,
            pltpu.SemaphoreType.DMA,
            pltpu.SemaphoreType.DMA,
            pltpu.SemaphoreType.DMA,
        ],
    )
    def gather_add(
        ys, yd, src, dst, out,
        idx_s, idx_d, buf_a, buf_b,
        sem_a0, sem_a1, sem_b0, sem_b1, sem_w0, sem_w1,
    ):
        wid = lax.axis_index("s") * info.num_cores + lax.axis_index("c")
        base0 = wid * e_per_w
        sems_a = (sem_a0, sem_a1)
        sems_b = (sem_b0, sem_b1)
        sems_w = (sem_w0, sem_w1)

        pltpu.sync_copy(src.at[pl.ds(base0, e_per_w)], idx_s)
        pltpu.sync_copy(dst.at[pl.ds(base0, e_per_w)], idx_d)

        def gathers(c, slot):
            off = c * _CH
            pltpu.async_copy(
                ys.at[idx_s.at[pl.ds(off, _CH)]], buf_a.at[slot], sems_a[slot]
            )
            pltpu.async_copy(
                yd.at[idx_d.at[pl.ds(off, _CH)]], buf_b.at[slot], sems_b[slot]
            )

        def wait_gathers(c, slot):
            off = c * _CH
            pltpu.make_async_copy(
                ys.at[idx_s.at[pl.ds(off, _CH)]], buf_a.at[slot], sems_a[slot]
            ).wait()
            pltpu.make_async_copy(
                yd.at[idx_d.at[pl.ds(off, _CH)]], buf_b.at[slot], sems_b[slot]
            ).wait()

        def writeback(c, slot):
            pltpu.async_copy(
                buf_a.at[slot], out.at[pl.ds(base0 + c * _CH, _CH)], sems_w[slot]
            )

        def wait_writeback(c, slot):
            pltpu.make_async_copy(
                buf_a.at[slot], out.at[pl.ds(base0 + c * _CH, _CH)], sems_w[slot]
            ).wait()

        def add_rows(slot):
            def body(i, carry):
                for u in range(2):
                    for j in range(HIDDEN // 16):
                        sl = pl.ds(j * 16, 16)
                        buf_a[slot, 2 * i + u, sl] = (
                            buf_a[slot, 2 * i + u, sl] + buf_b[slot, 2 * i + u, sl]
                        )
                return carry

            lax.fori_loop(0, _CH // 2, body, 0)

        gathers(0, 0)
        gathers(1, 1)

        def step(c, slot):
            wait_gathers(c, slot)
            add_rows(slot)

            @pl.when(c >= 2)
            def _():
                wait_writeback(c - 2, slot)

            writeback(c, slot)

            @pl.when(c + 2 < n_full)
            def _():
                gathers(c + 2, slot)

        def pair_body(p, carry):
            step(2 * p, 0)
            step(2 * p + 1, 1)
            return carry

        lax.fori_loop(0, n_full // 2, pair_body, 0)

        wait_writeback(n_full - 2, 0)
        wait_writeback(n_full - 1, 1)

        if has_tail:
            off = e_per_w - _CH
            a = pltpu.async_copy(
                ys.at[idx_s.at[pl.ds(off, _CH)]], buf_a.at[0], sem_a0
            )
            b = pltpu.async_copy(
                yd.at[idx_d.at[pl.ds(off, _CH)]], buf_b.at[0], sem_b0
            )
            a.wait()
            b.wait()
            add_rows(0)
            pltpu.sync_copy(buf_a.at[0], out.at[pl.ds(base0 + off, _CH)])

    return gather_add


_MLP_BLK = 2000


def _mlp_body(g_ref, attr_ref, we_ref, b1_ref, w2_ref, b2_ref, out_ref):
    h = (
        g_ref[...]
        + jnp.dot(attr_ref[...], we_ref[...], preferred_element_type=jnp.float32)
        + b1_ref[...]
    )
    h = jnp.maximum(h, 0.0)
    out_ref[...] = (
        jnp.dot(h, w2_ref[...], preferred_element_type=jnp.float32) + b2_ref[...]
    )


def _edge_mlp(g, edge_attr, we, b1, w2, b2):
    e = g.shape[0]
    d_edge = edge_attr.shape[1]
    grid = e // _MLP_BLK
    return pl.pallas_call(
        _mlp_body,
        grid=(grid,),
        in_specs=[
            pl.BlockSpec((_MLP_BLK, HIDDEN), lambda i: (i, 0)),
            pl.BlockSpec((_MLP_BLK, d_edge), lambda i: (i, 0)),
            pl.BlockSpec((d_edge, HIDDEN), lambda i: (0, 0)),
            pl.BlockSpec((1, HIDDEN), lambda i: (0, 0)),
            pl.BlockSpec((HIDDEN, OUT_DIM), lambda i: (0, 0)),
            pl.BlockSpec((1, OUT_DIM), lambda i: (0, 0)),
        ],
        out_specs=pl.BlockSpec((_MLP_BLK, OUT_DIM), lambda i: (i, 0)),
        out_shape=jax.ShapeDtypeStruct((e, OUT_DIM), jnp.float32),
    )(g, edge_attr, we, b1, w2, b2)


def kernel(x, edge_index, edge_attr, W1, b1, W2, b2):
    d = x.shape[1]
    ws = W1[:d]
    wd = W1[d : 2 * d]
    we = W1[2 * d :]
    src = edge_index[0].astype(jnp.int32)
    dst = edge_index[1].astype(jnp.int32)

    ys, yd = _project_nodes(x, ws, wd)
    g = _make_gather_add(edge_attr.shape[0])(ys, yd, src, dst)
    h_e = _edge_mlp(
        g,
        edge_attr,
        we,
        b1.reshape(1, -1),
        W2,
        b2.reshape(1, -1),
    )
    return (x, edge_index, h_e)
